# baseline (speedup 1.0000x reference)
import functools

import jax
import jax.numpy as jnp
from jax import lax
from jax.experimental import pallas as pl
from jax.experimental.pallas import tpu as pltpu
from jax.experimental.pallas import tpu_sc as plsc

_VOCAB = 100000
_EMB = 32
_MSG_LEN = 200
_H_DIM = 1024
_BATCH = 16384

_BM = 512
_NC = _MSG_LEN // 4

_NW = 32
_CHUNK = 512
_IDX_W = 128
_IDX_ROWS = _CHUNK // _IDX_W
_OUT_R = _CHUNK * _EMB // 128

_NSLICE = 8


def _sc_gather(table, idx2d):
    rows = idx2d.shape[0] * _IDX_W
    rows_w = rows // _NW
    n_chunk = rows_w // _CHUNK
    mesh = plsc.VectorSubcoreMesh(core_axis_name="c", subcore_axis_name="s")

    @functools.partial(
        pl.kernel,
        mesh=mesh,
        compiler_params=pltpu.CompilerParams(use_tc_tiling_on_sc=False),
        out_type=jax.ShapeDtypeStruct((rows * _EMB // 128, 128), jnp.float32),
        scratch_types=[
            pltpu.VMEM((_IDX_ROWS, _IDX_W), jnp.int32),
            pltpu.VMEM((2, _IDX_W, _EMB), jnp.float32),
            pltpu.VMEM((_OUT_R, 128), jnp.float32),
            pltpu.SemaphoreType.DMA,
            pltpu.SemaphoreType.DMA,
            pltpu.SemaphoreType.DMA,
        ],
    )
    def k(tab_hbm, idx_hbm, out_hbm, idx_v, ga, rows2_v, g0, g1, osem):
        wid = lax.axis_index("s") * 2 + lax.axis_index("c")
        base = wid * rows_w
        gsems = (g0, g1)

        def copy_block(j):
            gb = ga.at[j % 2]

            def copy4(n4, carry):
                n = pl.multiple_of(n4 * 4, 4)
                for dn in range(4):
                    for q in range(4):
                        for s in range(2):
                            v = gb[(n + dn) * 4 + q, pl.ds(16 * s, 16)]
                            rows2_v[32 * j + n + dn, pl.ds(32 * q + 16 * s, 16)] = v
                return carry

            lax.fori_loop(0, 8, copy4, 0)

        def body(ci, carry):
            row0 = pl.multiple_of(base + ci * _CHUNK, _CHUNK)
            irow0 = pl.multiple_of(row0 // _IDX_W, _IDX_ROWS)
            orow0 = pl.multiple_of(row0 * _EMB // 128, _OUT_R)
            pltpu.sync_copy(idx_hbm.at[pl.ds(irow0, _IDX_ROWS)], idx_v)
            gwaits = [None, None]
            for j in range(_IDX_ROWS):
                gwaits[j % 2] = pltpu.async_copy(
                    tab_hbm.at[idx_v.at[j]], ga.at[j % 2], gsems[j % 2]
                )
                if j == 1:
                    @pl.when(ci > 0)
                    def _():
                        pltpu.make_async_copy(
                            rows2_v, out_hbm.at[pl.ds(0, _OUT_R)], osem
                        ).wait()
                if j >= 1:
                    gwaits[(j - 1) % 2].wait()
                    copy_block(j - 1)
            gwaits[(_IDX_ROWS - 1) % 2].wait()
            copy_block(_IDX_ROWS - 1)
            pltpu.async_copy(rows2_v, out_hbm.at[pl.ds(orow0, _OUT_R)], osem)
            return carry

        lax.fori_loop(0, n_chunk, body, 0)
        pltpu.make_async_copy(rows2_v, out_hbm.at[pl.ds(0, _OUT_R)], osem).wait()

    return k(table, idx2d)


def _tc_mlp(fl2, w_bf, b2d):
    nb = fl2.shape[0] // (_NC * _BM)
    bsz = nb * _BM

    def body(f_ref, w_ref, b_ref, o_ref):
        accs = [jnp.zeros((_BM, _H_DIM), jnp.float32) for _ in range(4)]
        for c in range(_NC):
            f_c = f_ref[pl.ds(c * _BM, _BM), :].astype(jnp.bfloat16)
            w_c = w_ref[pl.ds(c * 128, 128), :]
            accs[c % 4] = accs[c % 4] + jnp.dot(
                f_c, w_c, preferred_element_type=jnp.float32
            )
        acc = (accs[0] + accs[1]) + (accs[2] + accs[3]) + b_ref[...]
        o_ref[...] = jnp.maximum(acc, 0.0)

    return pl.pallas_call(
        body,
        grid=(nb,),
        in_specs=[
            pl.BlockSpec((_NC * _BM, 128), lambda i: (i, 0)),
            pl.BlockSpec((_MSG_LEN * _EMB, _H_DIM), lambda i: (0, 0)),
            pl.BlockSpec((1, _H_DIM), lambda i: (0, 0)),
        ],
        out_specs=pl.BlockSpec((_BM, _H_DIM), lambda i: (i, 0)),
        out_shape=jax.ShapeDtypeStruct((bsz, _H_DIM), jnp.float32),
    )(fl2, w_bf, b2d)


def _tc_permute(x):
    bsz = x.shape[0]
    nb = bsz // _BM

    def body(x_ref, o_ref):
        xb = x_ref[...]
        o_ref[...] = (
            xb.reshape(_BM // 32, 32, _NC, 4)
            .transpose(2, 0, 1, 3)
            .reshape(_BM * _NC // 32, 128)
        )

    return pl.pallas_call(
        body,
        grid=(nb,),
        in_specs=[pl.BlockSpec((_BM, _MSG_LEN), lambda i: (i, 0))],
        out_specs=pl.BlockSpec((_BM * _NC // 32, 128), lambda i: (i, 0)),
        out_shape=jax.ShapeDtypeStruct((bsz * _MSG_LEN // _IDX_W, _IDX_W), jnp.int32),
    )(x)


def kernel(x, table, W, b):
    w_bf = W.astype(jnp.bfloat16)
    b2d = b.reshape(1, _H_DIM)
    bs = _BATCH // _NSLICE
    outs = []
    for s in range(_NSLICE):
        xs = lax.slice_in_dim(x, s * bs, (s + 1) * bs, axis=0)
        xp = _tc_permute(xs)
        fl2 = _sc_gather(table, xp)
        outs.append(_tc_mlp(fl2, w_bf, b2d))
    return lax.concatenate(outs, 0)


# speedup vs baseline: 1.0590x; 1.0590x over previous
import functools

import jax
import jax.numpy as jnp
from jax import lax
from jax.experimental import pallas as pl
from jax.experimental.pallas import tpu as pltpu
from jax.experimental.pallas import tpu_sc as plsc

_VOCAB = 100000
_EMB = 32
_MSG_LEN = 200
_H_DIM = 1024
_BATCH = 16384

_BM = 512
_NC = _MSG_LEN // 4

_NW = 32
_CHUNK = 1024
_IDX_W = 128
_IDX_ROWS = _CHUNK // _IDX_W
_OUT_R = _CHUNK * _EMB // 128

_NSLICE = 4


def _sc_gather(table, idx2d):
    rows = idx2d.shape[0] * _IDX_W
    rows_w = rows // _NW
    n_chunk = rows_w // _CHUNK
    mesh = plsc.VectorSubcoreMesh(core_axis_name="c", subcore_axis_name="s")

    @functools.partial(
        pl.kernel,
        mesh=mesh,
        compiler_params=pltpu.CompilerParams(use_tc_tiling_on_sc=False),
        out_type=jax.ShapeDtypeStruct((rows * _EMB // 128, 128), jnp.float32),
        scratch_types=[
            pltpu.VMEM((_IDX_ROWS, _IDX_W), jnp.int32),
            pltpu.VMEM((2, _IDX_W, _EMB), jnp.float32),
            pltpu.VMEM((_OUT_R, 128), jnp.float32),
            pltpu.SemaphoreType.DMA,
            pltpu.SemaphoreType.DMA,
            pltpu.SemaphoreType.DMA,
        ],
    )
    def k(tab_hbm, idx_hbm, out_hbm, idx_v, ga, rows2_v, g0, g1, osem):
        wid = lax.axis_index("s") * 2 + lax.axis_index("c")
        base = wid * rows_w
        gsems = (g0, g1)

        def copy_block(j):
            gb = ga.at[j % 2]

            def copy4(n4, carry):
                n = pl.multiple_of(n4 * 4, 4)
                for dn in range(4):
                    for q in range(4):
                        for s in range(2):
                            v = gb[(n + dn) * 4 + q, pl.ds(16 * s, 16)]
                            rows2_v[32 * j + n + dn, pl.ds(32 * q + 16 * s, 16)] = v
                return carry

            lax.fori_loop(0, 8, copy4, 0)

        def body(ci, carry):
            row0 = pl.multiple_of(base + ci * _CHUNK, _CHUNK)
            irow0 = pl.multiple_of(row0 // _IDX_W, _IDX_ROWS)
            orow0 = pl.multiple_of(row0 * _EMB // 128, _OUT_R)
            pltpu.sync_copy(idx_hbm.at[pl.ds(irow0, _IDX_ROWS)], idx_v)
            gwaits = [None, None]
            for j in range(_IDX_ROWS):
                gwaits[j % 2] = pltpu.async_copy(
                    tab_hbm.at[idx_v.at[j]], ga.at[j % 2], gsems[j % 2]
                )
                if j == 1:
                    @pl.when(ci > 0)
                    def _():
                        pltpu.make_async_copy(
                            rows2_v, out_hbm.at[pl.ds(0, _OUT_R)], osem
                        ).wait()
                if j >= 1:
                    gwaits[(j - 1) % 2].wait()
                    copy_block(j - 1)
            gwaits[(_IDX_ROWS - 1) % 2].wait()
            copy_block(_IDX_ROWS - 1)
            pltpu.async_copy(rows2_v, out_hbm.at[pl.ds(orow0, _OUT_R)], osem)
            return carry

        lax.fori_loop(0, n_chunk, body, 0)
        pltpu.make_async_copy(rows2_v, out_hbm.at[pl.ds(0, _OUT_R)], osem).wait()

    return k(table, idx2d)


def _tc_mlp(fl2, w_bf, b2d):
    nb = fl2.shape[0] // (_NC * _BM)
    bsz = nb * _BM

    def body(f_ref, w_ref, b_ref, o_ref):
        accs = [jnp.zeros((_BM, _H_DIM), jnp.float32) for _ in range(4)]
        for c in range(_NC):
            f_c = f_ref[pl.ds(c * _BM, _BM), :].astype(jnp.bfloat16)
            w_c = w_ref[pl.ds(c * 128, 128), :]
            accs[c % 4] = accs[c % 4] + jnp.dot(
                f_c, w_c, preferred_element_type=jnp.float32
            )
        acc = (accs[0] + accs[1]) + (accs[2] + accs[3]) + b_ref[...]
        o_ref[...] = jnp.maximum(acc, 0.0)

    return pl.pallas_call(
        body,
        grid=(nb,),
        in_specs=[
            pl.BlockSpec((_NC * _BM, 128), lambda i: (i, 0)),
            pl.BlockSpec((_MSG_LEN * _EMB, _H_DIM), lambda i: (0, 0)),
            pl.BlockSpec((1, _H_DIM), lambda i: (0, 0)),
        ],
        out_specs=pl.BlockSpec((_BM, _H_DIM), lambda i: (i, 0)),
        out_shape=jax.ShapeDtypeStruct((bsz, _H_DIM), jnp.float32),
    )(fl2, w_bf, b2d)


def _tc_permute(x):
    bsz = x.shape[0]
    nb = bsz // _BM

    def body(x_ref, o_ref):
        xb = x_ref[...]
        o_ref[...] = (
            xb.reshape(_BM // 32, 32, _NC, 4)
            .transpose(2, 0, 1, 3)
            .reshape(_BM * _NC // 32, 128)
        )

    return pl.pallas_call(
        body,
        grid=(nb,),
        in_specs=[pl.BlockSpec((_BM, _MSG_LEN), lambda i: (i, 0))],
        out_specs=pl.BlockSpec((_BM * _NC // 32, 128), lambda i: (i, 0)),
        out_shape=jax.ShapeDtypeStruct((bsz * _MSG_LEN // _IDX_W, _IDX_W), jnp.int32),
    )(x)


def kernel(x, table, W, b):
    w_bf = W.astype(jnp.bfloat16)
    b2d = b.reshape(1, _H_DIM)
    bs = _BATCH // _NSLICE
    outs = []
    for s in range(_NSLICE):
        xs = lax.slice_in_dim(x, s * bs, (s + 1) * bs, axis=0)
        xp = _tc_permute(xs)
        fl2 = _sc_gather(table, xp)
        outs.append(_tc_mlp(fl2, w_bf, b2d))
    return lax.concatenate(outs, 0)
